# baseline (speedup 1.0000x reference)
import functools

import jax
import jax.numpy as jnp
from jax import lax
from jax.experimental import pallas as pl
from jax.experimental.pallas import tpu as pltpu
from jax.experimental.pallas import tpu_sc as plsc

N_NODES = 10000
N_EDGES = 320000
D = 128

NC = 2
NS = 16

CHUNK = 128
TCH = 2560
E_PAD = TCH * CHUNK
CPT = TCH // (NC * NS)
CPP = 40

H_PAD = 10112
STRIPE = H_PAD // NS

_mesh = plsc.VectorSubcoreMesh(core_axis_name="c", subcore_axis_name="s",
                               num_cores=NC)


@functools.partial(
    pl.kernel,
    out_type=jax.ShapeDtypeStruct((NC, H_PAD, D), jnp.float32),
    mesh=_mesh,
    scratch_types=[
        pltpu.VMEM((CPP, CHUNK), jnp.int32),
        pltpu.VMEM((CPP, CHUNK), jnp.int32),
        pltpu.VMEM((CHUNK, D), jnp.float32),
        pltpu.VMEM((CHUNK, D), jnp.float32),
        pltpu.VMEM_SHARED((H_PAD, D), jnp.float32),
        pltpu.SemaphoreType.DMA,
        pltpu.SemaphoreType.DMA,
    ],
)
def _sc_segment_sum(x_hbm, src_hbm, dst_hbm, zeros_hbm, out_hbm,
                    src_v, dst_v, buf0, buf1, h_sh, sem0, sem1):
    c = lax.axis_index("c")
    s = lax.axis_index("s")
    tile_base = (c * NS + s) * CPT

    pltpu.sync_copy(zeros_hbm.at[pl.ds(s * STRIPE, STRIPE)],
                    h_sh.at[pl.ds(s * STRIPE, STRIPE)])
    plsc.subcore_barrier()

    def gather_start(j, buf, sem):
        pltpu.make_async_copy(x_hbm.at[src_v.at[j]], buf, sem).start()

    def gather_wait(buf, sem):
        pltpu.make_async_copy(x_hbm.at[src_v.at[0]], buf, sem).wait()

    def scatter_add(j, buf):
        pltpu.sync_copy(buf, h_sh.at[dst_v.at[j]], add=True)

    def one_pass(p, carry):
        row = pl.multiple_of(tile_base + p * CPP, 8)
        pltpu.sync_copy(src_hbm.at[pl.ds(row, CPP)], src_v)
        pltpu.sync_copy(dst_hbm.at[pl.ds(row, CPP)], dst_v)

        gather_start(0, buf0, sem0)

        def body(i, carry2):
            j0 = 2 * i
            gather_start(j0 + 1, buf1, sem1)
            gather_wait(buf0, sem0)
            scatter_add(j0, buf0)
            pl.when(j0 + 2 < CPP)(lambda: gather_start(j0 + 2, buf0, sem0))
            gather_wait(buf1, sem1)
            scatter_add(j0 + 1, buf1)
            return carry2

        lax.fori_loop(0, CPP // 2, body, 0)
        return carry

    lax.fori_loop(0, CPT // CPP, one_pass, 0)
    plsc.subcore_barrier()

    pltpu.sync_copy(h_sh.at[pl.ds(s * STRIPE, STRIPE)],
                    out_hbm.at[c, pl.ds(s * STRIPE, STRIPE)])


def _tc_body(h_ref, x_ref, wl_ref, ws_ref, b_ref, o_ref):
    h = h_ref[0] + h_ref[1]
    acc = jnp.dot(h, wl_ref[...], preferred_element_type=jnp.float32)
    acc = acc + jnp.dot(x_ref[...], ws_ref[...],
                        preferred_element_type=jnp.float32)
    o_ref[...] = acc + b_ref[...]


_BLK = 400


def _tc_linear(hpart, x, wlT, wsT, brow):
    return pl.pallas_call(
        _tc_body,
        grid=(N_NODES // _BLK,),
        in_specs=[
            pl.BlockSpec((NC, _BLK, D), lambda i: (0, i, 0)),
            pl.BlockSpec((_BLK, D), lambda i: (i, 0)),
            pl.BlockSpec((D, D), lambda i: (0, 0)),
            pl.BlockSpec((D, D), lambda i: (0, 0)),
            pl.BlockSpec((1, D), lambda i: (0, 0)),
        ],
        out_specs=pl.BlockSpec((_BLK, D), lambda i: (i, 0)),
        out_shape=jax.ShapeDtypeStruct((N_NODES, D), jnp.float32),
    )(hpart, x, wlT, wsT, brow)


def kernel(x, edge_index, W_lin, b_lin, W_self, b_self, bias):
    src = edge_index[0].astype(jnp.int32)
    dst = edge_index[1].astype(jnp.int32)
    pad = E_PAD - N_EDGES
    ar = jnp.arange(pad, dtype=jnp.int32)
    dummy_src = ar % N_NODES
    dummy_dst = N_NODES + ar % (H_PAD - N_NODES)
    src_p = jnp.concatenate([src, dummy_src]).reshape(TCH, CHUNK)
    dst_p = jnp.concatenate([dst, dummy_dst]).reshape(TCH, CHUNK)
    zeros = jnp.zeros((H_PAD, D), jnp.float32)

    hpart = _sc_segment_sum(x, src_p, dst_p, zeros)

    brow = (b_lin + b_self + bias).reshape(1, D)
    return _tc_linear(hpart, x, W_lin.T, W_self.T, brow)


# speedup vs baseline: 1.0672x; 1.0672x over previous
import functools

import jax
import jax.numpy as jnp
from jax import lax
from jax.experimental import pallas as pl
from jax.experimental.pallas import tpu as pltpu
from jax.experimental.pallas import tpu_sc as plsc

N_NODES = 10000
N_EDGES = 320000
D = 128

NC = 2
NS = 16

CHUNK = 128
TCH = 2560
E_PAD = TCH * CHUNK
CPT = TCH // (NC * NS)
CPP = 40

H_PAD = 10112
STRIPE = H_PAD // NS

_mesh = plsc.VectorSubcoreMesh(core_axis_name="c", subcore_axis_name="s",
                               num_cores=NC)


@functools.partial(
    pl.kernel,
    out_type=jax.ShapeDtypeStruct((NC, H_PAD, D), jnp.float32),
    mesh=_mesh,
    scratch_types=[
        pltpu.VMEM((CPP, CHUNK), jnp.int32),
        pltpu.VMEM((CPP, CHUNK), jnp.int32),
        pltpu.VMEM((CHUNK, D), jnp.float32),
        pltpu.VMEM((CHUNK, D), jnp.float32),
        pltpu.VMEM_SHARED((H_PAD, D), jnp.float32),
        pltpu.SemaphoreType.DMA,
        pltpu.SemaphoreType.DMA,
    ],
)
def _sc_segment_sum(x_hbm, src_hbm, dst_hbm, zeros_hbm, out_hbm,
                    src_v, dst_v, buf0, buf1, h_sh, sem0, sem1):
    c = lax.axis_index("c")
    s = lax.axis_index("s")
    tile_base = (c * NS + s) * CPT

    pltpu.sync_copy(zeros_hbm.at[pl.ds(s * STRIPE, STRIPE)],
                    h_sh.at[pl.ds(s * STRIPE, STRIPE)])
    plsc.subcore_barrier()

    def gather_start(j, buf, sem):
        pltpu.make_async_copy(x_hbm.at[src_v.at[j]], buf, sem).start()

    def gather_wait(buf, sem):
        pltpu.make_async_copy(x_hbm.at[src_v.at[0]], buf, sem).wait()

    def scatter_add(j, buf):
        pltpu.sync_copy(buf, h_sh.at[dst_v.at[j]], add=True)

    def one_pass(p, carry):
        row = pl.multiple_of(tile_base + p * CPP, 8)
        pltpu.sync_copy(src_hbm.at[pl.ds(row, CPP)], src_v)
        pltpu.sync_copy(dst_hbm.at[pl.ds(row, CPP)], dst_v)

        gather_start(0, buf0, sem0)

        def body(i, carry2):
            j0 = 2 * i
            gather_start(j0 + 1, buf1, sem1)
            gather_wait(buf0, sem0)
            scatter_add(j0, buf0)
            pl.when(j0 + 2 < CPP)(lambda: gather_start(j0 + 2, buf0, sem0))
            gather_wait(buf1, sem1)
            scatter_add(j0 + 1, buf1)
            return carry2

        lax.fori_loop(0, CPP // 2, body, 0)
        return carry

    lax.fori_loop(0, CPT // CPP, one_pass, 0)
    plsc.subcore_barrier()

    pltpu.sync_copy(h_sh.at[pl.ds(s * STRIPE, STRIPE)],
                    out_hbm.at[c, pl.ds(s * STRIPE, STRIPE)])


def _tc_body(h_ref, x_ref, wl_ref, ws_ref, b_ref, o_ref):
    h = h_ref[0] + h_ref[1]
    acc = jnp.dot(h, wl_ref[...], preferred_element_type=jnp.float32)
    acc = acc + jnp.dot(x_ref[...], ws_ref[...],
                        preferred_element_type=jnp.float32)
    o_ref[...] = acc + b_ref[...]


_BLK = 2000


def _tc_linear(hpart, x, wlT, wsT, brow):
    return pl.pallas_call(
        _tc_body,
        grid=(N_NODES // _BLK,),
        in_specs=[
            pl.BlockSpec((NC, _BLK, D), lambda i: (0, i, 0)),
            pl.BlockSpec((_BLK, D), lambda i: (i, 0)),
            pl.BlockSpec((D, D), lambda i: (0, 0)),
            pl.BlockSpec((D, D), lambda i: (0, 0)),
            pl.BlockSpec((1, D), lambda i: (0, 0)),
        ],
        out_specs=pl.BlockSpec((_BLK, D), lambda i: (i, 0)),
        out_shape=jax.ShapeDtypeStruct((N_NODES, D), jnp.float32),
    )(hpart, x, wlT, wsT, brow)


def kernel(x, edge_index, W_lin, b_lin, W_self, b_self, bias):
    src = edge_index[0].astype(jnp.int32)
    dst = edge_index[1].astype(jnp.int32)
    pad = E_PAD - N_EDGES
    ar = jnp.arange(pad, dtype=jnp.int32)
    dummy_src = ar % N_NODES
    dummy_dst = N_NODES + ar % (H_PAD - N_NODES)
    src_p = jnp.concatenate([src, dummy_src]).reshape(TCH, CHUNK)
    dst_p = jnp.concatenate([dst, dummy_dst]).reshape(TCH, CHUNK)
    zeros = jnp.zeros((H_PAD, D), jnp.float32)

    hpart = _sc_segment_sum(x, src_p, dst_p, zeros)

    brow = (b_lin + b_self + bias).reshape(1, D)
    return _tc_linear(hpart, x, W_lin.T, W_self.T, brow)


# speedup vs baseline: 1.0717x; 1.0042x over previous
import functools

import jax
import jax.numpy as jnp
from jax import lax
from jax.experimental import pallas as pl
from jax.experimental.pallas import tpu as pltpu
from jax.experimental.pallas import tpu_sc as plsc

N_NODES = 10000
N_EDGES = 320000
D = 128

NC = 2
NS = 16

CHUNK = 128
TCH = 2560
E_PAD = TCH * CHUNK
CPT = TCH // (NC * NS)
CPP = 40

H_PAD = 10112
STRIPE = H_PAD // NS

_mesh = plsc.VectorSubcoreMesh(core_axis_name="c", subcore_axis_name="s",
                               num_cores=NC)


@functools.partial(
    pl.kernel,
    out_type=jax.ShapeDtypeStruct((NC, H_PAD, D), jnp.float32),
    mesh=_mesh,
    scratch_types=[
        pltpu.VMEM((CPP, CHUNK), jnp.int32),
        pltpu.VMEM((CPP, CHUNK), jnp.int32),
        pltpu.VMEM((CHUNK, D), jnp.float32),
        pltpu.VMEM((CHUNK, D), jnp.float32),
        pltpu.VMEM_SHARED((H_PAD, D), jnp.float32),
        pltpu.SemaphoreType.DMA,
        pltpu.SemaphoreType.DMA,
    ],
)
def _sc_segment_sum(x_hbm, src_hbm, dst_hbm, zeros_hbm, out_hbm,
                    src_v, dst_v, buf0, buf1, h_sh, sem0, sem1):
    c = lax.axis_index("c")
    s = lax.axis_index("s")
    tile_base = (c * NS + s) * CPT

    pltpu.sync_copy(zeros_hbm, h_sh.at[pl.ds(s * STRIPE, STRIPE)])
    plsc.subcore_barrier()

    def gather_start(j, buf, sem):
        pltpu.make_async_copy(x_hbm.at[src_v.at[j]], buf, sem).start()

    def gather_wait(buf, sem):
        pltpu.make_async_copy(x_hbm.at[src_v.at[0]], buf, sem).wait()

    def scatter_add(j, buf):
        pltpu.sync_copy(buf, h_sh.at[dst_v.at[j]], add=True)

    def one_pass(p, carry):
        row = pl.multiple_of(tile_base + p * CPP, 8)
        pltpu.sync_copy(src_hbm.at[pl.ds(row, CPP)], src_v)
        pltpu.sync_copy(dst_hbm.at[pl.ds(row, CPP)], dst_v)

        gather_start(0, buf0, sem0)

        def body(i, carry2):
            j0 = 2 * i
            gather_start(j0 + 1, buf1, sem1)
            gather_wait(buf0, sem0)
            scatter_add(j0, buf0)
            pl.when(j0 + 2 < CPP)(lambda: gather_start(j0 + 2, buf0, sem0))
            gather_wait(buf1, sem1)
            scatter_add(j0 + 1, buf1)
            return carry2

        lax.fori_loop(0, CPP // 2, body, 0)
        return carry

    lax.fori_loop(0, CPT // CPP, one_pass, 0)
    plsc.subcore_barrier()

    pltpu.sync_copy(h_sh.at[pl.ds(s * STRIPE, STRIPE)],
                    out_hbm.at[c, pl.ds(s * STRIPE, STRIPE)])


_DNUMS = (((1,), (1,)), ((), ()))


def _tc_body(h_ref, x_ref, wl_ref, ws_ref, b_ref, o_ref):
    h = h_ref[0] + h_ref[1]
    acc = lax.dot_general(h, wl_ref[...], _DNUMS,
                          preferred_element_type=jnp.float32)
    acc = acc + lax.dot_general(x_ref[...], ws_ref[...], _DNUMS,
                                preferred_element_type=jnp.float32)
    o_ref[...] = acc + b_ref[...]


_BLK = 2000


def _tc_linear(hpart, x, wlT, wsT, brow):
    return pl.pallas_call(
        _tc_body,
        grid=(N_NODES // _BLK,),
        in_specs=[
            pl.BlockSpec((NC, _BLK, D), lambda i: (0, i, 0)),
            pl.BlockSpec((_BLK, D), lambda i: (i, 0)),
            pl.BlockSpec((D, D), lambda i: (0, 0)),
            pl.BlockSpec((D, D), lambda i: (0, 0)),
            pl.BlockSpec((1, D), lambda i: (0, 0)),
        ],
        out_specs=pl.BlockSpec((_BLK, D), lambda i: (i, 0)),
        out_shape=jax.ShapeDtypeStruct((N_NODES, D), jnp.float32),
    )(hpart, x, wlT, wsT, brow)


def kernel(x, edge_index, W_lin, b_lin, W_self, b_self, bias):
    src = edge_index[0].astype(jnp.int32)
    dst = edge_index[1].astype(jnp.int32)
    pad = E_PAD - N_EDGES
    ar = jnp.arange(pad, dtype=jnp.int32)
    dummy_src = ar % N_NODES
    dummy_dst = N_NODES + ar % (H_PAD - N_NODES)
    src_p = jnp.concatenate([src, dummy_src]).reshape(TCH, CHUNK)
    dst_p = jnp.concatenate([dst, dummy_dst]).reshape(TCH, CHUNK)
    zeros = jnp.zeros((STRIPE, D), jnp.float32)

    hpart = _sc_segment_sum(x, src_p, dst_p, zeros)

    brow = (b_lin + b_self + bias).reshape(1, D)
    return _tc_linear(hpart, x, W_lin, W_self, brow)


# speedup vs baseline: 1.0734x; 1.0016x over previous
import functools

import jax
import jax.numpy as jnp
from jax import lax
from jax.experimental import pallas as pl
from jax.experimental.pallas import tpu as pltpu
from jax.experimental.pallas import tpu_sc as plsc

N_NODES = 10000
N_EDGES = 320000
D = 128

NC = 2
NS = 16

CHUNK = 128
TCH = 2560
E_PAD = TCH * CHUNK
CPT = TCH // (NC * NS)
CPP = 40

H_PAD = 10112
STRIPE = H_PAD // NS

_mesh = plsc.VectorSubcoreMesh(core_axis_name="c", subcore_axis_name="s",
                               num_cores=NC)


@functools.partial(
    pl.kernel,
    out_type=jax.ShapeDtypeStruct((NC, H_PAD, D), jnp.float32),
    mesh=_mesh,
    scratch_types=[
        pltpu.VMEM((CPP, CHUNK), jnp.int32),
        pltpu.VMEM((CPP, CHUNK), jnp.int32),
        pltpu.VMEM((CHUNK, D), jnp.float32),
        pltpu.VMEM((CHUNK, D), jnp.float32),
        pltpu.VMEM_SHARED((H_PAD, D), jnp.float32),
        pltpu.SemaphoreType.DMA,
        pltpu.SemaphoreType.DMA,
    ],
)
def _sc_segment_sum(x_hbm, src_hbm, dst_hbm, zeros_hbm, out_hbm,
                    src_v, dst_v, buf0, buf1, h_sh, sem0, sem1):
    c = lax.axis_index("c")
    s = lax.axis_index("s")
    tile_base = (c * NS + s) * CPT

    pltpu.sync_copy(zeros_hbm, h_sh.at[pl.ds(s * STRIPE, STRIPE)])
    plsc.subcore_barrier()

    def gather_start(j, buf, sem):
        pltpu.make_async_copy(x_hbm.at[src_v.at[j]], buf, sem).start()

    def gather_wait(buf, sem):
        pltpu.make_async_copy(x_hbm.at[src_v.at[0]], buf, sem).wait()

    def scatter_add(j, buf):
        pltpu.sync_copy(buf, h_sh.at[dst_v.at[j]], add=True)

    def one_pass(p, carry):
        row = pl.multiple_of(tile_base + p * CPP, 8)
        pltpu.sync_copy(src_hbm.at[pl.ds(row, CPP)], src_v)
        pltpu.sync_copy(dst_hbm.at[pl.ds(row, CPP)], dst_v)

        gather_start(0, buf0, sem0)

        def body(i, carry2):
            j0 = 2 * i
            gather_start(j0 + 1, buf1, sem1)
            gather_wait(buf0, sem0)
            scatter_add(j0, buf0)
            pl.when(j0 + 2 < CPP)(lambda: gather_start(j0 + 2, buf0, sem0))
            gather_wait(buf1, sem1)
            scatter_add(j0 + 1, buf1)
            return carry2

        lax.fori_loop(0, CPP // 2, body, 0)
        return carry

    lax.fori_loop(0, CPT // CPP, one_pass, 0)
    plsc.subcore_barrier()

    pltpu.sync_copy(h_sh.at[pl.ds(s * STRIPE, STRIPE)],
                    out_hbm.at[c, pl.ds(s * STRIPE, STRIPE)])


_DNUMS = (((1,), (1,)), ((), ()))


def _tc_body(h_ref, x_ref, wl_ref, ws_ref, b_ref, o_ref):
    h = h_ref[0] + h_ref[1]
    acc = lax.dot_general(h, wl_ref[...], _DNUMS,
                          preferred_element_type=jnp.float32)
    acc = acc + lax.dot_general(x_ref[...], ws_ref[...], _DNUMS,
                                preferred_element_type=jnp.float32)
    o_ref[...] = acc + b_ref[...]


_BLK = 2000


def _tc_linear(hpart, x, wl, ws, brow):
    return pl.pallas_call(
        _tc_body,
        grid=(N_NODES // _BLK,),
        in_specs=[
            pl.BlockSpec((NC, _BLK, D), lambda i: (0, i, 0)),
            pl.BlockSpec((_BLK, D), lambda i: (i, 0)),
            pl.BlockSpec((D, D), lambda i: (0, 0)),
            pl.BlockSpec((D, D), lambda i: (0, 0)),
            pl.BlockSpec((1, D), lambda i: (0, 0)),
        ],
        out_specs=pl.BlockSpec((_BLK, D), lambda i: (i, 0)),
        out_shape=jax.ShapeDtypeStruct((N_NODES, D), jnp.float32),
    )(hpart, x, wl, ws, brow)


def kernel(x, edge_index, W_lin, b_lin, W_self, b_self, bias):
    src = edge_index[0].astype(jnp.int32)
    dst = edge_index[1].astype(jnp.int32)
    pad = E_PAD - N_EDGES
    ar = jnp.arange(pad, dtype=jnp.int32)
    dummy_src = ar % N_NODES
    dummy_dst = N_NODES + ar % (H_PAD - N_NODES)
    src_p = jnp.concatenate([src, dummy_src]).reshape(TCH, CHUNK)
    dst_p = jnp.concatenate([dst, dummy_dst]).reshape(TCH, CHUNK)
    zeros = jnp.zeros((STRIPE, D), jnp.float32)

    hpart = _sc_segment_sum(x, src_p, dst_p, zeros)

    brow = (b_lin + b_self + bias).reshape(1, D)
    return _tc_linear(hpart, x, W_lin, W_self, brow)
